# baseline (speedup 1.0000x reference)
import numpy as np
import jax
import jax.numpy as jnp
from jax import lax
from jax.experimental import pallas as pl
from jax.experimental.pallas import tpu as pltpu
from jax.experimental.pallas import tpu_sc as plsc

N = 10000
E = 320000
D_IN = 128
D_HID = 128
D_OUT = 64
K = 10

NPAD = 10240
NC, NS = 2, 16
DH = D_OUT // NC
SUB = 80
EPT = E // NS
NSUB = EPT // SUB
RPT = NPAD // NS
NBLK = RPT // SUB


def _jackson_damp():
    k = np.arange(K + 1, dtype=np.float64)
    c = np.pi / (K + 2)
    damp = ((K + 2 - k) * np.sin(c) * np.cos(k * c)
            + np.cos(c) * np.sin(k * c)) / ((K + 2) * np.sin(c))
    return damp.astype(np.float32)


_DAMP = _jackson_damp()


def _mlp_body(x_ref, w1t_ref, b1_ref, w2t_ref, b2_ref, h_ref):
    h1 = jnp.dot(x_ref[...], w1t_ref[...], preferred_element_type=jnp.float32)
    h1 = h1 + b1_ref[...][None, :]
    h1 = jnp.where(h1 > 0, h1, 0.01 * h1)
    h2 = jnp.dot(h1, w2t_ref[...], preferred_element_type=jnp.float32)
    h_ref[...] = h2 + b2_ref[...][None, :]


def _mlp(x, w1t, b1, w2t, b2):
    R = 1024
    return pl.pallas_call(
        _mlp_body,
        grid=(NPAD // R,),
        in_specs=[
            pl.BlockSpec((R, D_IN), lambda i: (i, 0)),
            pl.BlockSpec((D_IN, D_HID), lambda i: (0, 0)),
            pl.BlockSpec((D_HID,), lambda i: (0,)),
            pl.BlockSpec((D_HID, D_OUT), lambda i: (0, 0)),
            pl.BlockSpec((D_OUT,), lambda i: (0,)),
        ],
        out_specs=pl.BlockSpec((R, D_OUT), lambda i: (i, 0)),
        out_shape=jax.ShapeDtypeStruct((NPAD, D_OUT), jnp.float32),
    )(x, w1t, b1, w2t, b2)


def _sc_params():
    return pltpu.CompilerParams(needs_layout_passes=False, use_tc_tiling_on_sc=False)


def _deg_body(col_hbm, w_hbm, z_hbm, degp_hbm, colv, wv, deg_sh, ssem):
    c = lax.axis_index("c")
    s = lax.axis_index("s")
    wid = c * NS + s
    nsub_half = NSUB // 2
    eb = wid * nsub_half
    pltpu.sync_copy(col_hbm.at[pl.ds(eb, nsub_half)], colv)
    pltpu.sync_copy(w_hbm.at[pl.ds(eb, nsub_half)], wv)
    pltpu.sync_copy(z_hbm.at[pl.ds(s * RPT, RPT)], deg_sh.at[pl.ds(s * RPT, RPT)])
    plsc.subcore_barrier()

    for k in range(4):
        pltpu.async_copy(wv.at[k], deg_sh.at[colv.at[k]], ssem, add=True)

    def chunk(k, carry):
        pltpu.async_copy(wv.at[k], deg_sh.at[colv.at[k]], ssem, add=True)
        pltpu.make_async_copy(wv.at[0], deg_sh.at[colv.at[0]], ssem).wait()
        return carry

    lax.fori_loop(4, nsub_half, chunk, 0)
    for k in range(4):
        pltpu.make_async_copy(wv.at[0], deg_sh.at[colv.at[0]], ssem).wait()
    plsc.subcore_barrier()
    pltpu.sync_copy(deg_sh.at[pl.ds(s * RPT, RPT)],
                    degp_hbm.at[c, pl.ds(s * RPT, RPT)])


def _deg(col2, w2, zpad):
    mesh = plsc.VectorSubcoreMesh(core_axis_name="c", subcore_axis_name="s")
    f = pl.kernel(
        _deg_body,
        out_type=jax.ShapeDtypeStruct((NC, NPAD), jnp.float32),
        mesh=mesh,
        compiler_params=_sc_params(),
        scratch_types=[
            pltpu.VMEM((NSUB // 2, SUB), jnp.int32),
            pltpu.VMEM((NSUB // 2, SUB), jnp.float32),
            pltpu.VMEM_SHARED((NPAD,), jnp.float32),
            pltpu.SemaphoreType.DMA,
        ],
    )
    return f(col2, w2, zpad)


def _prep_body(degp_ref, h_ref, dinv_ref, h2_ref):
    deg = degp_ref[0, :] + degp_ref[1, :]
    dinv = jnp.where(deg > 0, lax.rsqrt(jnp.maximum(deg, 1e-12)), 0.0)
    dinv_ref[...] = jnp.broadcast_to(dinv[:, None], dinv_ref.shape)
    h = h_ref[...]
    h2_ref[0] = h[:, :DH]
    h2_ref[1] = h[:, DH:]


def _prep(degp, h):
    R = 1024
    return pl.pallas_call(
        _prep_body,
        grid=(NPAD // R,),
        in_specs=[
            pl.BlockSpec((NC, R), lambda i: (0, i)),
            pl.BlockSpec((R, D_OUT), lambda i: (i, 0)),
        ],
        out_specs=[
            pl.BlockSpec((R, 16), lambda i: (i, 0)),
            pl.BlockSpec((NC, R, DH), lambda i: (0, i, 0)),
        ],
        out_shape=[
            jax.ShapeDtypeStruct((NPAD, 16), jnp.float32),
            jax.ShapeDtypeStruct((NC, NPAD, DH), jnp.float32),
        ],
    )(degp, h)


def _wexp_body(w_ref, o_ref):
    o_ref[...] = jnp.broadcast_to(w_ref[...], o_ref.shape)


def _wexp(w):
    R = 8000
    return pl.pallas_call(
        _wexp_body,
        grid=(E // R,),
        in_specs=[pl.BlockSpec((R, 1), lambda i: (i, 0))],
        out_specs=pl.BlockSpec((R, 16), lambda i: (i, 0)),
        out_shape=jax.ShapeDtypeStruct((E, 16), jnp.float32),
    )(w.reshape(E, 1))


def _cheb_body(h2_hbm, dinv_hbm, row_hbm, col_hbm, w_hbm, z_hbm, coef_hbm,
               ab_hbm, out_hbm, tx_hbm,
               rowv, colv, w0, w1, w2, w3, w4, b0, b1, b2, b3, b4,
               dinv_v, coefv, abv, acc_sh, sv_sh,
               g0, g1, g2, g3, g4, s0, s1, s2, s3, s4):
    c = lax.axis_index("c")
    s = lax.axis_index("s")
    bufs = (b0, b1, b2, b3, b4)
    wbufs = (w0, w1, w2, w3, w4)
    gsems = (g0, g1, g2, g3, g4)
    ssems = (s0, s1, s2, s3, s4)

    pltpu.sync_copy(row_hbm.at[pl.ds(s * NSUB, NSUB)], rowv)
    pltpu.sync_copy(col_hbm.at[pl.ds(s * NSUB, NSUB)], colv)
    pltpu.sync_copy(dinv_hbm.at[pl.ds(s * RPT, RPT)], dinv_v)
    pltpu.sync_copy(coef_hbm, coefv)
    pltpu.sync_copy(ab_hbm, abv)
    c0b = coefv[0, pl.ds(0, 16)]

    def initblk(b, carry):
        base = s * RPT + b * SUB
        pltpu.sync_copy(h2_hbm.at[c, pl.ds(base, SUB)], b0)

        def irow(r, carry2):
            dv = dinv_v[b * SUB + r, pl.ds(0, 16)]
            for q in range(DH // 16):
                hq = b0[r, pl.ds(q * 16, 16)]
                b1[r, pl.ds(q * 16, 16)] = hq * dv
                b2[r, pl.ds(q * 16, 16)] = hq * c0b
            return carry2

        lax.fori_loop(0, SUB, irow, 0)
        pltpu.sync_copy(b1, sv_sh.at[pl.ds(base, SUB)])
        pltpu.sync_copy(b0, tx_hbm.at[0, c, pl.ds(base, SUB)])
        pltpu.sync_copy(b0, tx_hbm.at[1, c, pl.ds(base, SUB)])
        pltpu.sync_copy(b2, out_hbm.at[c, pl.ds(base, SUB)])
        pltpu.sync_copy(z_hbm.at[pl.ds(base, SUB)], acc_sh.at[pl.ds(base, SUB)])
        return carry

    lax.fori_loop(0, NBLK, initblk, 0)
    plsc.subcore_barrier()

    def scale(bi, k):
        buf = bufs[bi]
        wb = wbufs[bi]

        def grp(g, carry):
            for i in range(16):
                e = g * 16 + i
                bwi = wb[e, pl.ds(0, 16)]
                for q in range(DH // 16):
                    buf[e, pl.ds(q * 16, 16)] = buf[e, pl.ds(q * 16, 16)] * bwi
            return carry

        lax.fori_loop(0, SUB // 16, grp, 0)

    def gissue(a, bi):
        pltpu.async_copy(sv_sh.at[rowv.at[a]], bufs[bi], gsems[bi])

    def gwait(a, bi):
        pltpu.make_async_copy(sv_sh.at[rowv.at[0]], bufs[bi], gsems[bi]).wait()
        pltpu.sync_copy(w_hbm.at[pl.ds(s * EPT + a * SUB, SUB)], wbufs[bi])

    def sissue(a, bi):
        pltpu.async_copy(bufs[bi], acc_sh.at[colv.at[a]], ssems[bi], add=True)

    def swait(bi):
        pltpu.make_async_copy(bufs[bi], acc_sh.at[colv.at[0]], ssems[bi]).wait()

    def hop(k, carry):
        gissue(0, 0)
        gissue(1, 1)
        for a in range(3):
            gwait(a, a)
            scale(a, a)
            gissue(a + 2, (a + 2) % 5)
            sissue(a, a)

        def body(kk, carry2):
            for jj in range(5):
                a = 3 + 5 * kk + jj
                bi = (3 + jj) % 5
                gwait(a, bi)
                scale(bi, a)
                swait((bi + 2) % 5)
                gissue(a + 2, (bi + 2) % 5)
                sissue(a, bi)
            return carry2

        lax.fori_loop(0, (NSUB - 5) // 5, body, 0)
        for a in (NSUB - 2, NSUB - 1):
            bi = a % 5
            gwait(a, bi)
            scale(bi, a)
            swait((bi + 2) % 5)
            sissue(a, bi)
        for bi in ((NSUB - 3) % 5, (NSUB - 2) % 5, (NSUB - 1) % 5):
            swait(bi)
        plsc.subcore_barrier()

        k2 = k % 2
        alpha = abv[k, pl.ds(0, 16)]
        beta = abv[16 + k, pl.ds(0, 16)]
        coefk = coefv[k, pl.ds(0, 16)]

        def blk(b, carry2):
            base = s * RPT + b * SUB
            pltpu.sync_copy(acc_sh.at[pl.ds(base, SUB)], b0)
            pltpu.sync_copy(tx_hbm.at[k2, c, pl.ds(base, SUB)], b1)
            pltpu.sync_copy(out_hbm.at[c, pl.ds(base, SUB)], b2)

            def crow(r, carry3):
                dv = dinv_v[b * SUB + r, pl.ds(0, 16)]
                for q in range(DH // 16):
                    p = b0[r, pl.ds(q * 16, 16)] * dv
                    t2 = alpha * p - beta * b1[r, pl.ds(q * 16, 16)]
                    b1[r, pl.ds(q * 16, 16)] = t2
                    b2[r, pl.ds(q * 16, 16)] = b2[r, pl.ds(q * 16, 16)] + coefk * t2
                    b0[r, pl.ds(q * 16, 16)] = t2 * dv
                return carry3

            lax.fori_loop(0, SUB, crow, 0)
            pltpu.sync_copy(b1, tx_hbm.at[k2, c, pl.ds(base, SUB)])
            pltpu.sync_copy(b2, out_hbm.at[c, pl.ds(base, SUB)])
            pltpu.sync_copy(b0, sv_sh.at[pl.ds(base, SUB)])
            pltpu.sync_copy(z_hbm.at[pl.ds(base, SUB)], acc_sh.at[pl.ds(base, SUB)])
            return carry2

        lax.fori_loop(0, NBLK, blk, 0)
        plsc.subcore_barrier()
        return carry

    lax.fori_loop(1, K + 1, hop, 0)


def _cheb(h2, dinv, row2, col2, wflat, zpad2, coefs, ab):
    mesh = plsc.VectorSubcoreMesh(core_axis_name="c", subcore_axis_name="s")
    f = pl.kernel(
        _cheb_body,
        out_type=[
            jax.ShapeDtypeStruct((NC, NPAD, DH), jnp.float32),
            jax.ShapeDtypeStruct((2, NC, NPAD, DH), jnp.float32),
        ],
        mesh=mesh,
        compiler_params=_sc_params(),
        scratch_types=(
            [pltpu.VMEM((NSUB, SUB), jnp.int32),
             pltpu.VMEM((NSUB, SUB), jnp.int32)]
            + [pltpu.VMEM((SUB, 16), jnp.float32)] * 5
            + [pltpu.VMEM((SUB, DH), jnp.float32)] * 5
            + [pltpu.VMEM((RPT, 16), jnp.float32),
               pltpu.VMEM((16, 16), jnp.float32),
               pltpu.VMEM((32, 16), jnp.float32)]
            + [pltpu.VMEM_SHARED((NPAD, DH), jnp.float32)] * 2
            + [pltpu.SemaphoreType.DMA] * 10
        ),
    )
    return f(h2, dinv, row2, col2, wflat, zpad2, coefs, ab)


def kernel(x, edge_index, edge_weight, W1, b1, W2, b2, cheb_coef):
    row2 = edge_index[0].reshape(E // SUB, SUB)
    col2 = edge_index[1].reshape(E // SUB, SUB)
    ew2 = edge_weight.reshape(E // SUB, SUB)
    h = _mlp(x, W1.T, b1, W2.T, b2)

    zpad = jnp.zeros((NPAD,), jnp.float32)
    zpad2 = jnp.zeros((NPAD, DH), jnp.float32)
    degp = _deg(col2, ew2, zpad)

    coefs = jnp.zeros((16,), jnp.float32).at[:K + 1].set(
        cheb_coef * jnp.asarray(_DAMP))
    coefexp = jnp.broadcast_to(coefs[:, None], (16, 16))
    dinvexp, h2 = _prep(degp, h)
    wexp = _wexp(edge_weight)
    alpha = np.zeros((16,), np.float32)
    beta = np.zeros((16,), np.float32)
    alpha[1] = 1.0
    alpha[2:K + 1] = 2.0
    beta[2:K + 1] = 1.0
    abexp = jnp.broadcast_to(
        jnp.asarray(np.concatenate([alpha, beta]))[:, None], (32, 16))

    out2, _tx = _cheb(h2, dinvexp, row2, col2, wexp, zpad2, coefexp, abexp)
    return jnp.concatenate([out2[0, :N], out2[1, :N]], axis=1)


# speedup vs baseline: 1.6056x; 1.6056x over previous
import numpy as np
import jax
import jax.numpy as jnp
from jax import lax
from jax.experimental import pallas as pl
from jax.experimental.pallas import tpu as pltpu
from jax.experimental.pallas import tpu_sc as plsc

N = 10000
E = 320000
D_IN = 128
D_HID = 128
D_OUT = 64
K = 10

NPAD = 10240
NC, NS = 2, 16
DH = D_OUT // NC
SUB = 80
EPT = E // NS
NSUB = EPT // SUB
RPT = NPAD // NS
NBLK = RPT // SUB


def _jackson_damp():
    k = np.arange(K + 1, dtype=np.float64)
    c = np.pi / (K + 2)
    damp = ((K + 2 - k) * np.sin(c) * np.cos(k * c)
            + np.cos(c) * np.sin(k * c)) / ((K + 2) * np.sin(c))
    return damp.astype(np.float32)


_DAMP = _jackson_damp()


def _mlp_body(x_ref, w1t_ref, b1_ref, w2t_ref, b2_ref, h_ref):
    h1 = jnp.dot(x_ref[...], w1t_ref[...], preferred_element_type=jnp.float32)
    h1 = h1 + b1_ref[...][None, :]
    h1 = jnp.where(h1 > 0, h1, 0.01 * h1)
    h2 = jnp.dot(h1, w2t_ref[...], preferred_element_type=jnp.float32)
    h_ref[...] = h2 + b2_ref[...][None, :]


def _mlp(x, w1t, b1, w2t, b2):
    R = 1024
    return pl.pallas_call(
        _mlp_body,
        grid=(NPAD // R,),
        in_specs=[
            pl.BlockSpec((R, D_IN), lambda i: (i, 0)),
            pl.BlockSpec((D_IN, D_HID), lambda i: (0, 0)),
            pl.BlockSpec((D_HID,), lambda i: (0,)),
            pl.BlockSpec((D_HID, D_OUT), lambda i: (0, 0)),
            pl.BlockSpec((D_OUT,), lambda i: (0,)),
        ],
        out_specs=pl.BlockSpec((R, D_OUT), lambda i: (i, 0)),
        out_shape=jax.ShapeDtypeStruct((NPAD, D_OUT), jnp.float32),
    )(x, w1t, b1, w2t, b2)


def _sc_params():
    return pltpu.CompilerParams(needs_layout_passes=False, use_tc_tiling_on_sc=False)


def _deg_body(col_hbm, w_hbm, z_hbm, degp_hbm, colv, wv, deg_sh, ssem):
    c = lax.axis_index("c")
    s = lax.axis_index("s")
    wid = c * NS + s
    nsub_half = NSUB // 2
    eb = wid * nsub_half
    pltpu.sync_copy(col_hbm.at[pl.ds(eb, nsub_half)], colv)
    pltpu.sync_copy(w_hbm.at[pl.ds(eb, nsub_half)], wv)
    pltpu.sync_copy(z_hbm.at[pl.ds(s * RPT, RPT)], deg_sh.at[pl.ds(s * RPT, RPT)])
    plsc.subcore_barrier()

    for k in range(4):
        pltpu.async_copy(wv.at[k], deg_sh.at[colv.at[k]], ssem, add=True)

    def chunk(k, carry):
        pltpu.async_copy(wv.at[k], deg_sh.at[colv.at[k]], ssem, add=True)
        pltpu.make_async_copy(wv.at[0], deg_sh.at[colv.at[0]], ssem).wait()
        return carry

    lax.fori_loop(4, nsub_half, chunk, 0)
    for k in range(4):
        pltpu.make_async_copy(wv.at[0], deg_sh.at[colv.at[0]], ssem).wait()
    plsc.subcore_barrier()
    pltpu.sync_copy(deg_sh.at[pl.ds(s * RPT, RPT)],
                    degp_hbm.at[c, pl.ds(s * RPT, RPT)])


def _deg(col2, w2, zpad):
    mesh = plsc.VectorSubcoreMesh(core_axis_name="c", subcore_axis_name="s")
    f = pl.kernel(
        _deg_body,
        out_type=jax.ShapeDtypeStruct((NC, NPAD), jnp.float32),
        mesh=mesh,
        compiler_params=_sc_params(),
        scratch_types=[
            pltpu.VMEM((NSUB // 2, SUB), jnp.int32),
            pltpu.VMEM((NSUB // 2, SUB), jnp.float32),
            pltpu.VMEM_SHARED((NPAD,), jnp.float32),
            pltpu.SemaphoreType.DMA,
        ],
    )
    return f(col2, w2, zpad)


def _prep_body(degp_ref, h_ref, dinv_ref, h2_ref):
    deg = degp_ref[0, :] + degp_ref[1, :]
    dinv = jnp.where(deg > 0, lax.rsqrt(jnp.maximum(deg, 1e-12)), 0.0)
    dinv_ref[...] = jnp.broadcast_to(dinv[:, None], dinv_ref.shape)
    h = h_ref[...]
    h2_ref[0] = h[:, :DH]
    h2_ref[1] = h[:, DH:]


def _prep(degp, h):
    R = 1024
    return pl.pallas_call(
        _prep_body,
        grid=(NPAD // R,),
        in_specs=[
            pl.BlockSpec((NC, R), lambda i: (0, i)),
            pl.BlockSpec((R, D_OUT), lambda i: (i, 0)),
        ],
        out_specs=[
            pl.BlockSpec((R, 16), lambda i: (i, 0)),
            pl.BlockSpec((NC, R, DH), lambda i: (0, i, 0)),
        ],
        out_shape=[
            jax.ShapeDtypeStruct((NPAD, 16), jnp.float32),
            jax.ShapeDtypeStruct((NC, NPAD, DH), jnp.float32),
        ],
    )(degp, h)


def _wexp_body(w_ref, o_ref):
    o_ref[...] = jnp.broadcast_to(w_ref[...], o_ref.shape)


def _wexp(w):
    R = 8000
    return pl.pallas_call(
        _wexp_body,
        grid=(E // R,),
        in_specs=[pl.BlockSpec((R, 1), lambda i: (i, 0))],
        out_specs=pl.BlockSpec((R, 16), lambda i: (i, 0)),
        out_shape=jax.ShapeDtypeStruct((E, 16), jnp.float32),
    )(w.reshape(E, 1))


def _cheb_body(h2_hbm, dinv_hbm, row_hbm, col_hbm, w_hbm, z_hbm, coef_hbm,
               ab_hbm, out_hbm, tx_hbm,
               rowv, colv, w0, w1, w2, w3, w4, b0, b1, b2, b3, b4,
               dinv_v, coefv, abv, acc_sh, sv_sh,
               g0, g1, g2, g3, g4, s0, s1, s2, s3, s4,
               m0, m1, m2, m3, m4):
    c = lax.axis_index("c")
    s = lax.axis_index("s")
    bufs = (b0, b1, b2, b3, b4)
    wbufs = (w0, w1, w2, w3, w4)
    gsems = (g0, g1, g2, g3, g4)
    ssems = (s0, s1, s2, s3, s4)
    wsems = (m0, m1, m2, m3, m4)

    pltpu.sync_copy(row_hbm.at[pl.ds(s * NSUB, NSUB)], rowv)
    pltpu.sync_copy(col_hbm.at[pl.ds(s * NSUB, NSUB)], colv)
    pltpu.sync_copy(dinv_hbm.at[pl.ds(s * RPT, RPT)], dinv_v)
    pltpu.sync_copy(coef_hbm, coefv)
    pltpu.sync_copy(ab_hbm, abv)
    c0b = coefv[0, pl.ds(0, 16)]

    def initblk(b, carry):
        base = s * RPT + b * SUB
        pltpu.sync_copy(h2_hbm.at[c, pl.ds(base, SUB)], b0)

        def irow(r, carry2):
            dv = dinv_v[b * SUB + r, pl.ds(0, 16)]
            for q in range(DH // 16):
                hq = b0[r, pl.ds(q * 16, 16)]
                b1[r, pl.ds(q * 16, 16)] = hq * dv
                b2[r, pl.ds(q * 16, 16)] = hq * c0b
            return carry2

        lax.fori_loop(0, SUB, irow, 0)
        pltpu.sync_copy(b1, sv_sh.at[pl.ds(base, SUB)])
        pltpu.sync_copy(b0, tx_hbm.at[0, c, pl.ds(base, SUB)])
        pltpu.sync_copy(b0, tx_hbm.at[1, c, pl.ds(base, SUB)])
        pltpu.sync_copy(b2, out_hbm.at[c, pl.ds(base, SUB)])
        pltpu.sync_copy(z_hbm.at[pl.ds(base, SUB)], acc_sh.at[pl.ds(base, SUB)])
        return carry

    lax.fori_loop(0, NBLK, initblk, 0)
    plsc.subcore_barrier()

    def scale(bi, k):
        buf = bufs[bi]
        wb = wbufs[bi]

        def grp(g, carry):
            for i in range(16):
                e = g * 16 + i
                bwi = wb[e, pl.ds(0, 16)]
                for q in range(DH // 16):
                    buf[e, pl.ds(q * 16, 16)] = buf[e, pl.ds(q * 16, 16)] * bwi
            return carry

        lax.fori_loop(0, SUB // 16, grp, 0)

    def gissue(a, bi):
        pltpu.async_copy(sv_sh.at[rowv.at[a]], bufs[bi], gsems[bi])
        pltpu.async_copy(w_hbm.at[pl.ds(s * EPT + a * SUB, SUB)], wbufs[bi],
                         wsems[bi])

    def gwait(a, bi):
        pltpu.make_async_copy(sv_sh.at[rowv.at[0]], bufs[bi], gsems[bi]).wait()
        pltpu.make_async_copy(w_hbm.at[pl.ds(0, SUB)], wbufs[bi],
                              wsems[bi]).wait()

    def sissue(a, bi):
        pltpu.async_copy(bufs[bi], acc_sh.at[colv.at[a]], ssems[bi], add=True)

    def swait(bi):
        pltpu.make_async_copy(bufs[bi], acc_sh.at[colv.at[0]], ssems[bi]).wait()

    def hop(k, carry):
        gissue(0, 0)
        gissue(1, 1)
        for a in range(3):
            gwait(a, a)
            scale(a, a)
            gissue(a + 2, (a + 2) % 5)
            sissue(a, a)

        def body(kk, carry2):
            for jj in range(5):
                a = 3 + 5 * kk + jj
                bi = (3 + jj) % 5
                gwait(a, bi)
                scale(bi, a)
                swait((bi + 2) % 5)
                gissue(a + 2, (bi + 2) % 5)
                sissue(a, bi)
            return carry2

        lax.fori_loop(0, (NSUB - 5) // 5, body, 0)
        for a in (NSUB - 2, NSUB - 1):
            bi = a % 5
            gwait(a, bi)
            scale(bi, a)
            swait((bi + 2) % 5)
            sissue(a, bi)
        for bi in ((NSUB - 3) % 5, (NSUB - 2) % 5, (NSUB - 1) % 5):
            swait(bi)
        plsc.subcore_barrier()

        k2 = k % 2
        alpha = abv[k, pl.ds(0, 16)]
        beta = abv[16 + k, pl.ds(0, 16)]
        coefk = coefv[k, pl.ds(0, 16)]

        def blk(b, carry2):
            base = s * RPT + b * SUB
            pltpu.sync_copy(acc_sh.at[pl.ds(base, SUB)], b0)
            pltpu.sync_copy(tx_hbm.at[k2, c, pl.ds(base, SUB)], b1)
            pltpu.sync_copy(out_hbm.at[c, pl.ds(base, SUB)], b2)

            def crow(r, carry3):
                dv = dinv_v[b * SUB + r, pl.ds(0, 16)]
                for q in range(DH // 16):
                    p = b0[r, pl.ds(q * 16, 16)] * dv
                    t2 = alpha * p - beta * b1[r, pl.ds(q * 16, 16)]
                    b1[r, pl.ds(q * 16, 16)] = t2
                    b2[r, pl.ds(q * 16, 16)] = b2[r, pl.ds(q * 16, 16)] + coefk * t2
                    b0[r, pl.ds(q * 16, 16)] = t2 * dv
                return carry3

            lax.fori_loop(0, SUB, crow, 0)
            pltpu.sync_copy(b1, tx_hbm.at[k2, c, pl.ds(base, SUB)])
            pltpu.sync_copy(b2, out_hbm.at[c, pl.ds(base, SUB)])
            pltpu.sync_copy(b0, sv_sh.at[pl.ds(base, SUB)])
            pltpu.sync_copy(z_hbm.at[pl.ds(base, SUB)], acc_sh.at[pl.ds(base, SUB)])
            return carry2

        lax.fori_loop(0, NBLK, blk, 0)
        plsc.subcore_barrier()
        return carry

    lax.fori_loop(1, K + 1, hop, 0)


def _cheb(h2, dinv, row2, col2, wflat, zpad2, coefs, ab):
    mesh = plsc.VectorSubcoreMesh(core_axis_name="c", subcore_axis_name="s")
    f = pl.kernel(
        _cheb_body,
        out_type=[
            jax.ShapeDtypeStruct((NC, NPAD, DH), jnp.float32),
            jax.ShapeDtypeStruct((2, NC, NPAD, DH), jnp.float32),
        ],
        mesh=mesh,
        compiler_params=_sc_params(),
        scratch_types=(
            [pltpu.VMEM((NSUB, SUB), jnp.int32),
             pltpu.VMEM((NSUB, SUB), jnp.int32)]
            + [pltpu.VMEM((SUB, 16), jnp.float32)] * 5
            + [pltpu.VMEM((SUB, DH), jnp.float32)] * 5
            + [pltpu.VMEM((RPT, 16), jnp.float32),
               pltpu.VMEM((16, 16), jnp.float32),
               pltpu.VMEM((32, 16), jnp.float32)]
            + [pltpu.VMEM_SHARED((NPAD, DH), jnp.float32)] * 2
            + [pltpu.SemaphoreType.DMA] * 15
        ),
    )
    return f(h2, dinv, row2, col2, wflat, zpad2, coefs, ab)


def kernel(x, edge_index, edge_weight, W1, b1, W2, b2, cheb_coef):
    row2 = edge_index[0].reshape(E // SUB, SUB)
    col2 = edge_index[1].reshape(E // SUB, SUB)
    ew2 = edge_weight.reshape(E // SUB, SUB)
    h = _mlp(x, W1.T, b1, W2.T, b2)

    zpad = jnp.zeros((NPAD,), jnp.float32)
    zpad2 = jnp.zeros((NPAD, DH), jnp.float32)
    degp = _deg(col2, ew2, zpad)

    coefs = jnp.zeros((16,), jnp.float32).at[:K + 1].set(
        cheb_coef * jnp.asarray(_DAMP))
    coefexp = jnp.broadcast_to(coefs[:, None], (16, 16))
    dinvexp, h2 = _prep(degp, h)
    wexp = _wexp(edge_weight)
    alpha = np.zeros((16,), np.float32)
    beta = np.zeros((16,), np.float32)
    alpha[1] = 1.0
    alpha[2:K + 1] = 2.0
    beta[2:K + 1] = 1.0
    abexp = jnp.broadcast_to(
        jnp.asarray(np.concatenate([alpha, beta]))[:, None], (32, 16))

    out2, _tx = _cheb(h2, dinvexp, row2, col2, wexp, zpad2, coefexp, abexp)
    return jnp.concatenate([out2[0, :N], out2[1, :N]], axis=1)


# speedup vs baseline: 1.8946x; 1.1800x over previous
import numpy as np
import jax
import jax.numpy as jnp
from jax import lax
from jax.experimental import pallas as pl
from jax.experimental.pallas import tpu as pltpu
from jax.experimental.pallas import tpu_sc as plsc

N = 10000
E = 320000
D_IN = 128
D_HID = 128
D_OUT = 64
K = 10

NPAD = 10240
NC, NS = 2, 16
DH = D_OUT // NC
SUB = 128
EPAD = 327680
EPT = EPAD // NS
NSUB = EPT // SUB
RPT = NPAD // NS
NBLK = RPT // SUB


def _jackson_damp():
    k = np.arange(K + 1, dtype=np.float64)
    c = np.pi / (K + 2)
    damp = ((K + 2 - k) * np.sin(c) * np.cos(k * c)
            + np.cos(c) * np.sin(k * c)) / ((K + 2) * np.sin(c))
    return damp.astype(np.float32)


_DAMP = _jackson_damp()


def _mlp_body(x_ref, w1t_ref, b1_ref, w2t_ref, b2_ref, h_ref):
    h1 = jnp.dot(x_ref[...], w1t_ref[...], preferred_element_type=jnp.float32)
    h1 = h1 + b1_ref[...][None, :]
    h1 = jnp.where(h1 > 0, h1, 0.01 * h1)
    h2 = jnp.dot(h1, w2t_ref[...], preferred_element_type=jnp.float32)
    h_ref[...] = h2 + b2_ref[...][None, :]


def _mlp(x, w1t, b1, w2t, b2):
    R = 1024
    return pl.pallas_call(
        _mlp_body,
        grid=(NPAD // R,),
        in_specs=[
            pl.BlockSpec((R, D_IN), lambda i: (i, 0)),
            pl.BlockSpec((D_IN, D_HID), lambda i: (0, 0)),
            pl.BlockSpec((D_HID,), lambda i: (0,)),
            pl.BlockSpec((D_HID, D_OUT), lambda i: (0, 0)),
            pl.BlockSpec((D_OUT,), lambda i: (0,)),
        ],
        out_specs=pl.BlockSpec((R, D_OUT), lambda i: (i, 0)),
        out_shape=jax.ShapeDtypeStruct((NPAD, D_OUT), jnp.float32),
    )(x, w1t, b1, w2t, b2)


def _sc_params():
    return pltpu.CompilerParams(needs_layout_passes=False, use_tc_tiling_on_sc=False)


def _deg_body(col_hbm, w_hbm, z_hbm, degp_hbm, colv, wv, deg_sh, ssem):
    c = lax.axis_index("c")
    s = lax.axis_index("s")
    wid = c * NS + s
    nsub_half = NSUB // 2
    eb = wid * nsub_half
    pltpu.sync_copy(col_hbm.at[pl.ds(eb, nsub_half)], colv)
    pltpu.sync_copy(w_hbm.at[pl.ds(eb, nsub_half)], wv)
    pltpu.sync_copy(z_hbm.at[pl.ds(s * RPT, RPT)], deg_sh.at[pl.ds(s * RPT, RPT)])
    plsc.subcore_barrier()

    for k in range(4):
        pltpu.async_copy(wv.at[k], deg_sh.at[colv.at[k]], ssem, add=True)

    def chunk(k, carry):
        pltpu.async_copy(wv.at[k], deg_sh.at[colv.at[k]], ssem, add=True)
        pltpu.make_async_copy(wv.at[0], deg_sh.at[colv.at[0]], ssem).wait()
        return carry

    lax.fori_loop(4, nsub_half, chunk, 0)
    for k in range(4):
        pltpu.make_async_copy(wv.at[0], deg_sh.at[colv.at[0]], ssem).wait()
    plsc.subcore_barrier()
    pltpu.sync_copy(deg_sh.at[pl.ds(s * RPT, RPT)],
                    degp_hbm.at[c, pl.ds(s * RPT, RPT)])


def _deg(col2, w2, zpad):
    mesh = plsc.VectorSubcoreMesh(core_axis_name="c", subcore_axis_name="s")
    f = pl.kernel(
        _deg_body,
        out_type=jax.ShapeDtypeStruct((NC, NPAD), jnp.float32),
        mesh=mesh,
        compiler_params=_sc_params(),
        scratch_types=[
            pltpu.VMEM((NSUB // 2, SUB), jnp.int32),
            pltpu.VMEM((NSUB // 2, SUB), jnp.float32),
            pltpu.VMEM_SHARED((NPAD,), jnp.float32),
            pltpu.SemaphoreType.DMA,
        ],
    )
    return f(col2, w2, zpad)


def _prep_body(degp_ref, h_ref, dinv_ref, h2_ref):
    deg = degp_ref[0, :] + degp_ref[1, :]
    dinv = jnp.where(deg > 0, lax.rsqrt(jnp.maximum(deg, 1e-12)), 0.0)
    dinv_ref[...] = jnp.broadcast_to(dinv[:, None], dinv_ref.shape)
    h = h_ref[...]
    h2_ref[0] = h[:, :DH]
    h2_ref[1] = h[:, DH:]


def _prep(degp, h):
    R = 1024
    return pl.pallas_call(
        _prep_body,
        grid=(NPAD // R,),
        in_specs=[
            pl.BlockSpec((NC, R), lambda i: (0, i)),
            pl.BlockSpec((R, D_OUT), lambda i: (i, 0)),
        ],
        out_specs=[
            pl.BlockSpec((R, 16), lambda i: (i, 0)),
            pl.BlockSpec((NC, R, DH), lambda i: (0, i, 0)),
        ],
        out_shape=[
            jax.ShapeDtypeStruct((NPAD, 16), jnp.float32),
            jax.ShapeDtypeStruct((NC, NPAD, DH), jnp.float32),
        ],
    )(degp, h)


def _wexp_body(w_ref, o_ref):
    o_ref[...] = jnp.broadcast_to(w_ref[...], o_ref.shape)


def _wexp(w):
    R = 8192
    return pl.pallas_call(
        _wexp_body,
        grid=(EPAD // R,),
        in_specs=[pl.BlockSpec((R, 1), lambda i: (i, 0))],
        out_specs=pl.BlockSpec((R, 16), lambda i: (i, 0)),
        out_shape=jax.ShapeDtypeStruct((EPAD, 16), jnp.float32),
    )(w.reshape(EPAD, 1))


def _cheb_body(h2_hbm, dinv_hbm, row_hbm, col_hbm, w_hbm, z_hbm, coef_hbm,
               ab_hbm, out_hbm, tx_hbm,
               rowv, colv, w0, w1, w2, w3, w4, b0, b1, b2, b3, b4,
               dinv_v, coefv, abv, acc_sh, sv_sh,
               g0, g1, g2, g3, g4, s0, s1, s2, s3, s4,
               m0, m1, m2, m3, m4):
    c = lax.axis_index("c")
    s = lax.axis_index("s")
    bufs = (b0, b1, b2, b3, b4)
    wbufs = (w0, w1, w2, w3, w4)
    gsems = (g0, g1, g2, g3, g4)
    ssems = (s0, s1, s2, s3, s4)
    wsems = (m0, m1, m2, m3, m4)

    pltpu.sync_copy(row_hbm.at[pl.ds(s * NSUB, NSUB)], rowv)
    pltpu.sync_copy(col_hbm.at[pl.ds(s * NSUB, NSUB)], colv)
    pltpu.sync_copy(dinv_hbm.at[pl.ds(s * RPT, RPT)], dinv_v)
    pltpu.sync_copy(coef_hbm, coefv)
    pltpu.sync_copy(ab_hbm, abv)
    c0b = coefv[0, pl.ds(0, 16)]

    def initblk(b, carry):
        base = s * RPT + b * SUB
        pltpu.sync_copy(h2_hbm.at[c, pl.ds(base, SUB)], b0)

        def irow(r, carry2):
            dv = dinv_v[b * SUB + r, pl.ds(0, 16)]
            for q in range(DH // 16):
                hq = b0[r, pl.ds(q * 16, 16)]
                b1[r, pl.ds(q * 16, 16)] = hq * dv
                b2[r, pl.ds(q * 16, 16)] = hq * c0b
            return carry2

        lax.fori_loop(0, SUB, irow, 0)
        pltpu.sync_copy(b1, sv_sh.at[pl.ds(base, SUB)])
        pltpu.sync_copy(b0, tx_hbm.at[0, c, pl.ds(base, SUB)])
        pltpu.sync_copy(b0, tx_hbm.at[1, c, pl.ds(base, SUB)])
        pltpu.sync_copy(b2, out_hbm.at[c, pl.ds(base, SUB)])
        pltpu.sync_copy(z_hbm.at[pl.ds(base, SUB)], acc_sh.at[pl.ds(base, SUB)])
        return carry

    lax.fori_loop(0, NBLK, initblk, 0)
    plsc.subcore_barrier()

    def scale(bi, k):
        buf = bufs[bi]
        wb = wbufs[bi]

        def grp(g, carry):
            for i in range(16):
                e = g * 16 + i
                bwi = wb[e, pl.ds(0, 16)]
                for q in range(DH // 16):
                    buf[e, pl.ds(q * 16, 16)] = buf[e, pl.ds(q * 16, 16)] * bwi
            return carry

        lax.fori_loop(0, SUB // 16, grp, 0)

    def gissue(a, bi):
        pltpu.async_copy(sv_sh.at[rowv.at[a]], bufs[bi], gsems[bi])
        pltpu.async_copy(w_hbm.at[pl.ds(s * EPT + a * SUB, SUB)], wbufs[bi],
                         wsems[bi])

    def gwait(a, bi):
        pltpu.make_async_copy(sv_sh.at[rowv.at[0]], bufs[bi], gsems[bi]).wait()
        pltpu.make_async_copy(w_hbm.at[pl.ds(0, SUB)], wbufs[bi],
                              wsems[bi]).wait()

    def sissue(a, bi):
        pltpu.async_copy(bufs[bi], acc_sh.at[colv.at[a]], ssems[bi], add=True)

    def swait(bi):
        pltpu.make_async_copy(bufs[bi], acc_sh.at[colv.at[0]], ssems[bi]).wait()

    def hop(k, carry):
        gissue(0, 0)
        gissue(1, 1)
        for a in range(3):
            gwait(a, a)
            scale(a, a)
            gissue(a + 2, (a + 2) % 5)
            sissue(a, a)

        def body(kk, carry2):
            for jj in range(5):
                a = 3 + 5 * kk + jj
                bi = (3 + jj) % 5
                gwait(a, bi)
                scale(bi, a)
                swait((bi + 2) % 5)
                gissue(a + 2, (bi + 2) % 5)
                sissue(a, bi)
            return carry2

        lax.fori_loop(0, (NSUB - 5) // 5, body, 0)
        for a in (NSUB - 2, NSUB - 1):
            bi = a % 5
            gwait(a, bi)
            scale(bi, a)
            swait((bi + 2) % 5)
            sissue(a, bi)
        for bi in ((NSUB - 3) % 5, (NSUB - 2) % 5, (NSUB - 1) % 5):
            swait(bi)
        plsc.subcore_barrier()

        k2 = k % 2
        alpha = abv[k, pl.ds(0, 16)]
        beta = abv[16 + k, pl.ds(0, 16)]
        coefk = coefv[k, pl.ds(0, 16)]

        def blk(b, carry2):
            base = s * RPT + b * SUB
            pltpu.sync_copy(acc_sh.at[pl.ds(base, SUB)], b0)
            pltpu.sync_copy(tx_hbm.at[k2, c, pl.ds(base, SUB)], b1)
            pltpu.sync_copy(out_hbm.at[c, pl.ds(base, SUB)], b2)

            def crow(r, carry3):
                dv = dinv_v[b * SUB + r, pl.ds(0, 16)]
                for q in range(DH // 16):
                    p = b0[r, pl.ds(q * 16, 16)] * dv
                    t2 = alpha * p - beta * b1[r, pl.ds(q * 16, 16)]
                    b1[r, pl.ds(q * 16, 16)] = t2
                    b2[r, pl.ds(q * 16, 16)] = b2[r, pl.ds(q * 16, 16)] + coefk * t2
                    b0[r, pl.ds(q * 16, 16)] = t2 * dv
                return carry3

            lax.fori_loop(0, SUB, crow, 0)
            pltpu.sync_copy(b1, tx_hbm.at[k2, c, pl.ds(base, SUB)])
            pltpu.sync_copy(b2, out_hbm.at[c, pl.ds(base, SUB)])
            pltpu.sync_copy(b0, sv_sh.at[pl.ds(base, SUB)])
            pltpu.sync_copy(z_hbm.at[pl.ds(base, SUB)], acc_sh.at[pl.ds(base, SUB)])
            return carry2

        lax.fori_loop(0, NBLK, blk, 0)
        plsc.subcore_barrier()
        return carry

    lax.fori_loop(1, K + 1, hop, 0)


def _cheb(h2, dinv, row2, col2, wflat, zpad2, coefs, ab):
    mesh = plsc.VectorSubcoreMesh(core_axis_name="c", subcore_axis_name="s")
    f = pl.kernel(
        _cheb_body,
        out_type=[
            jax.ShapeDtypeStruct((NC, NPAD, DH), jnp.float32),
            jax.ShapeDtypeStruct((2, NC, NPAD, DH), jnp.float32),
        ],
        mesh=mesh,
        compiler_params=_sc_params(),
        scratch_types=(
            [pltpu.VMEM((NSUB, SUB), jnp.int32),
             pltpu.VMEM((NSUB, SUB), jnp.int32)]
            + [pltpu.VMEM((SUB, 16), jnp.float32)] * 5
            + [pltpu.VMEM((SUB, DH), jnp.float32)] * 5
            + [pltpu.VMEM((RPT, 16), jnp.float32),
               pltpu.VMEM((16, 16), jnp.float32),
               pltpu.VMEM((32, 16), jnp.float32)]
            + [pltpu.VMEM_SHARED((NPAD, DH), jnp.float32)] * 2
            + [pltpu.SemaphoreType.DMA] * 15
        ),
    )
    return f(h2, dinv, row2, col2, wflat, zpad2, coefs, ab)


def kernel(x, edge_index, edge_weight, W1, b1, W2, b2, cheb_coef):
    pad = EPAD - E
    rowp = jnp.concatenate([edge_index[0], jnp.zeros((pad,), jnp.int32)])
    colp = jnp.concatenate([edge_index[1], jnp.zeros((pad,), jnp.int32)])
    ewp = jnp.concatenate([edge_weight, jnp.zeros((pad,), jnp.float32)])
    row2 = rowp.reshape(EPAD // SUB, SUB)
    col2 = colp.reshape(EPAD // SUB, SUB)
    ew2 = ewp.reshape(EPAD // SUB, SUB)
    h = _mlp(x, W1.T, b1, W2.T, b2)

    zpad = jnp.zeros((NPAD,), jnp.float32)
    zpad2 = jnp.zeros((NPAD, DH), jnp.float32)
    degp = _deg(col2, ew2, zpad)

    coefs = jnp.zeros((16,), jnp.float32).at[:K + 1].set(
        cheb_coef * jnp.asarray(_DAMP))
    coefexp = jnp.broadcast_to(coefs[:, None], (16, 16))
    dinvexp, h2 = _prep(degp, h)
    wexp = _wexp(ewp)
    alpha = np.zeros((16,), np.float32)
    beta = np.zeros((16,), np.float32)
    alpha[1] = 1.0
    alpha[2:K + 1] = 2.0
    beta[2:K + 1] = 1.0
    abexp = jnp.broadcast_to(
        jnp.asarray(np.concatenate([alpha, beta]))[:, None], (32, 16))

    out2, _tx = _cheb(h2, dinvexp, row2, col2, wexp, zpad2, coefexp, abexp)
    return jnp.concatenate([out2[0, :N], out2[1, :N]], axis=1)
